# baseline (speedup 1.0000x reference)
import functools

import numpy as np
import jax
import jax.numpy as jnp
from jax import lax
from jax.experimental import pallas as pl
from jax.experimental.pallas import tpu as pltpu
from jax.experimental.pallas import tpu_sc as plsc

N = 10000
E = 320000
H = 128
T = 5

_NC = 2
_NS = 16
_NW = _NC * _NS
_EW = E // _NW
_BE = 40
_NBLK = _EW // _BE
_NP = 10240
_RPT = _NP // _NS
_LG = H // 16

_mesh = plsc.VectorSubcoreMesh(core_axis_name="c", subcore_axis_name="s")

_SIG = np.concatenate([
    np.stack([32 * g + np.arange(16), 32 * g + 16 + np.arange(16)], 1).reshape(-1)
    for g in range(H // 32)
])


def _edge_body(src_h, dst_h, xa_h, xb_h, wq_h, zeros_h, out_h,
               sidx0, sidx1, didx0, didx1, didx2, didx3,
               ra0, ra1, rb0, rb1, rw0, rw1,
               rt0, rt1, s_sh, gsem0, gsem1, ssem0, ssem1):
    c = lax.axis_index("c")
    s = lax.axis_index("s")
    wid = s * _NC + c
    base = wid * _EW

    sidx = (sidx0, sidx1)
    didx = (didx0, didx1, didx2, didx3)
    ra = (ra0, ra1)
    rb = (rb0, rb1)
    rw = (rw0, rw1)
    rt = (rt0, rt1)
    gsem = (gsem0, gsem1)
    ssem = (ssem0, ssem1)

    pltpu.sync_copy(zeros_h, s_sh.at[pl.ds(s * _RPT, _RPT)])
    plsc.subcore_barrier()

    def start_block(b, p, j):
        off = base + b * _BE
        pltpu.sync_copy(src_h.at[pl.ds(off, _BE)], sidx[p])
        pltpu.sync_copy(dst_h.at[pl.ds(off, _BE)], didx[j])
        pltpu.async_copy(xa_h.at[didx[j]], ra[p], gsem[p])
        pltpu.async_copy(xb_h.at[sidx[p]], rb[p], gsem[p])
        pltpu.async_copy(wq_h.at[pl.ds(off, _BE)], rw[p], gsem[p])

    def wait_scatter(p, j):
        pltpu.make_async_copy(rt[p], s_sh.at[didx[j]], ssem[p]).wait()

    def do_phase(b, p, j, first=False, last=False):
        pltpu.make_async_copy(xa_h.at[didx[j]], ra[p], gsem[p]).wait()
        pltpu.make_async_copy(xb_h.at[sidx[p]], rb[p], gsem[p]).wait()
        pltpu.make_async_copy(wq_h.at[pl.ds(0, _BE)], rw[p], gsem[p]).wait()
        if not first:
            wait_scatter(p, (j + 2) % 4)

        cra, crb, crw, crt = ra[p], rb[p], rw[p], rt[p]

        def edge(e, cc):
            _f32 = lambda v: lax.bitcast_convert_type(v, jnp.float32)
            m = jnp.int32(-65536)
            for g in range(H // 32):
                vw = crw[e, pl.ds(16 * g, 16)]
                for half in range(2):
                    fw = _f32(vw << 16) if half == 0 else _f32(vw & m)
                    sl = pl.ds(32 * g + 16 * half, 16)
                    z = cra[e, sl] + crb[e, sl] + fw
                    u = jnp.exp(z + z)
                    crt[e, sl] = 1.0 - 2.0 / (u + 1.0)
            return cc

        lax.fori_loop(0, _BE, edge, 0)
        pltpu.async_copy(rt[p], s_sh.at[didx[j]], ssem[p], add=True)
        if not last:
            start_block(b + 2, p, (j + 2) % 4)

    start_block(0, 0, 0)
    start_block(1, 1, 1)
    do_phase(0, 0, 0, first=True)
    do_phase(1, 1, 1, first=True)

    def quad(k, carry):
        b0 = 2 + 4 * k
        do_phase(b0, 0, 2)
        do_phase(b0 + 1, 1, 3)
        do_phase(b0 + 2, 0, 0)
        do_phase(b0 + 3, 1, 1)
        return carry

    lax.fori_loop(0, (_NBLK - 6) // 4, quad, 0)
    do_phase(_NBLK - 4, 0, 2)
    do_phase(_NBLK - 3, 1, 3)
    do_phase(_NBLK - 2, 0, 0, last=True)
    do_phase(_NBLK - 1, 1, 1, last=True)
    wait_scatter(0, 0)
    wait_scatter(1, 1)

    plsc.subcore_barrier()
    pltpu.sync_copy(s_sh.at[pl.ds(s * _RPT, _RPT)],
                    out_h.at[c, pl.ds(s * _RPT, _RPT)])


_edge_call = functools.partial(
    pl.kernel, _edge_body,
    out_type=jax.ShapeDtypeStruct((_NC, _NP, H), jnp.float32),
    mesh=_mesh,
    scratch_types=[
        pltpu.VMEM((_BE,), jnp.int32),
        pltpu.VMEM((_BE,), jnp.int32),
        pltpu.VMEM((_BE,), jnp.int32),
        pltpu.VMEM((_BE,), jnp.int32),
        pltpu.VMEM((_BE,), jnp.int32),
        pltpu.VMEM((_BE,), jnp.int32),
        pltpu.VMEM((_BE, H), jnp.float32),
        pltpu.VMEM((_BE, H), jnp.float32),
        pltpu.VMEM((_BE, H), jnp.float32),
        pltpu.VMEM((_BE, H), jnp.float32),
        pltpu.VMEM((_BE, H // 2), jnp.int32),
        pltpu.VMEM((_BE, H // 2), jnp.int32),
        pltpu.VMEM((_BE, H), jnp.float32),
        pltpu.VMEM((_BE, H), jnp.float32),
        pltpu.VMEM_SHARED((_NP, H), jnp.float32),
        pltpu.SemaphoreType.DMA,
        pltpu.SemaphoreType.DMA,
        pltpu.SemaphoreType.DMA,
        pltpu.SemaphoreType.DMA,
    ],
)()


def _cnt_body(dst_h, ones_h, zeros_h, out_h, didx, ones_v, c_sh):
    c = lax.axis_index("c")
    s = lax.axis_index("s")
    wid = s * _NC + c
    base = wid * _EW

    pltpu.sync_copy(zeros_h, c_sh.at[pl.ds(s * _RPT, _RPT)])
    pltpu.sync_copy(ones_h, ones_v)
    plsc.subcore_barrier()

    def blk(b, carry):
        off = base + b * _BE
        pltpu.sync_copy(dst_h.at[pl.ds(off, _BE)], didx)
        pltpu.sync_copy(ones_v, c_sh.at[didx], add=True)
        return carry

    lax.fori_loop(0, _NBLK, blk, 0)
    plsc.subcore_barrier()
    pltpu.sync_copy(c_sh.at[pl.ds(s * _RPT, _RPT)],
                    out_h.at[c, pl.ds(s * _RPT, _RPT)])


_cnt_call = functools.partial(
    pl.kernel, _cnt_body,
    out_type=jax.ShapeDtypeStruct((_NC, _NP, 16), jnp.float32),
    mesh=_mesh,
    scratch_types=[
        pltpu.VMEM((_BE,), jnp.int32),
        pltpu.VMEM((_BE, 16), jnp.float32),
        pltpu.VMEM_SHARED((_NP, 16), jnp.float32),
    ],
)()


_BEW = 4000
_BN = 2000


def _wq_body(w_ref, wg_ref, bg_ref, o_ref):
    o_ref[...] = (
        jnp.dot(w_ref[...], wg_ref[...], preferred_element_type=jnp.float32)
        + bg_ref[...]
    ).astype(jnp.bfloat16)


def _wq_call(w, wg1w, bg1):
    return pl.pallas_call(
        _wq_body,
        grid=(E // _BEW,),
        in_specs=[
            pl.BlockSpec((_BEW, H), lambda i: (i, 0)),
            pl.BlockSpec((H, H), lambda i: (0, 0)),
            pl.BlockSpec((1, H), lambda i: (0, 0)),
        ],
        out_specs=pl.BlockSpec((_BEW, H), lambda i: (i, 0)),
        out_shape=jax.ShapeDtypeStruct((E, H), jnp.bfloat16),
    )(w, wg1w, bg1)


def _k0_body(x01_ref, we1_ref, be1_ref, we2_ref, be2_ref, th_ref, wf1t_ref,
             bf1_ref, wg1a_ref, wg1b_ref, wo_ref, bo_ref,
             x_ref, xa_ref, xb_ref, tf_ref, y0_ref):
    h1 = jnp.tanh(x01_ref[...] * we1_ref[...] + be1_ref[...])
    x = jnp.dot(h1, we2_ref[...], preferred_element_type=jnp.float32) + be2_ref[...]
    x_ref[...] = x
    xa_ref[...] = jnp.dot(x, wg1a_ref[...], preferred_element_type=jnp.float32)
    xb_ref[...] = jnp.dot(x, wg1b_ref[...], preferred_element_type=jnp.float32)
    tf_ref[...] = (
        jnp.dot(th_ref[...], wf1t_ref[...], preferred_element_type=jnp.float32)
        + bf1_ref[...]
    )
    y0_ref[...] = jnp.dot(x, wo_ref[...], preferred_element_type=jnp.float32) + bo_ref[...]


def _k0_call(x01, we1, be1, we2, be2, theta, wf1t, bf1, wg1a, wg1b, wo, bo):
    full = lambda i: (0, 0)
    blk = lambda i: (i, 0)
    return pl.pallas_call(
        _k0_body,
        grid=(N // _BN,),
        in_specs=[
            pl.BlockSpec((_BN, 1), blk),
            pl.BlockSpec((1, H), full),
            pl.BlockSpec((1, H), full),
            pl.BlockSpec((H, H), full),
            pl.BlockSpec((1, H), full),
            pl.BlockSpec((_BN, H), blk),
            pl.BlockSpec((H, H), full),
            pl.BlockSpec((1, H), full),
            pl.BlockSpec((H, H), full),
            pl.BlockSpec((H, H), full),
            pl.BlockSpec((H, 1), full),
            pl.BlockSpec((1, 1), full),
        ],
        out_specs=[
            pl.BlockSpec((_BN, H), blk),
            pl.BlockSpec((_BN, H), blk),
            pl.BlockSpec((_BN, H), blk),
            pl.BlockSpec((_BN, H), blk),
            pl.BlockSpec((_BN, 1), blk),
        ],
        out_shape=[
            jax.ShapeDtypeStruct((N, H), jnp.float32),
            jax.ShapeDtypeStruct((N, H), jnp.float32),
            jax.ShapeDtypeStruct((N, H), jnp.float32),
            jax.ShapeDtypeStruct((N, H), jnp.float32),
            jax.ShapeDtypeStruct((N, 1), jnp.float32),
        ],
    )(x01, we1, be1, we2, be2, theta, wf1t, bf1, wg1a, wg1b, wo, bo)


def _k1_body(x_ref, tf_ref, s0_ref, s1_ref, c0_ref, c1_ref, dt_ref,
             wf1x_ref, wf2_ref, bf2_ref, wg2_ref, bg2_ref, wg1a_ref,
             wg1b_ref, wo_ref, bo_ref,
             xn_ref, xa_ref, xb_ref, y_ref):
    x = x_ref[...]
    h = jnp.tanh(
        jnp.dot(x, wf1x_ref[...], preferred_element_type=jnp.float32) + tf_ref[...]
    )
    self_h = jnp.dot(h, wf2_ref[...], preferred_element_type=jnp.float32) + bf2_ref[...]
    s = s0_ref[...] + s1_ref[...]
    cnt = c0_ref[:, :1] + c1_ref[:, :1]
    agg = (
        jnp.dot(s, wg2_ref[...], preferred_element_type=jnp.float32)
        + cnt * bg2_ref[...]
    )
    xn = x + dt_ref[0, 0] * (self_h + agg)
    xn_ref[...] = xn
    xa_ref[...] = jnp.dot(xn, wg1a_ref[...], preferred_element_type=jnp.float32)
    xb_ref[...] = jnp.dot(xn, wg1b_ref[...], preferred_element_type=jnp.float32)
    y_ref[...] = jnp.dot(xn, wo_ref[...], preferred_element_type=jnp.float32) + bo_ref[...]


def _k1_call(x, tf, s0, s1, c0, c1, dt, wf1x, wf2, bf2, wg2, bg2, wg1a,
             wg1b, wo, bo):
    full = lambda i: (0, 0)
    blk = lambda i: (i, 0)
    return pl.pallas_call(
        _k1_body,
        grid=(N // _BN,),
        in_specs=[
            pl.BlockSpec((_BN, H), blk),
            pl.BlockSpec((_BN, H), blk),
            pl.BlockSpec((_BN, H), blk),
            pl.BlockSpec((_BN, H), blk),
            pl.BlockSpec((_BN, 16), blk),
            pl.BlockSpec((_BN, 16), blk),
            pl.BlockSpec((1, 1), full),
            pl.BlockSpec((H, H), full),
            pl.BlockSpec((H, H), full),
            pl.BlockSpec((1, H), full),
            pl.BlockSpec((H, H), full),
            pl.BlockSpec((1, H), full),
            pl.BlockSpec((H, H), full),
            pl.BlockSpec((H, H), full),
            pl.BlockSpec((H, 1), full),
            pl.BlockSpec((1, 1), full),
        ],
        out_specs=[
            pl.BlockSpec((_BN, H), blk),
            pl.BlockSpec((_BN, H), blk),
            pl.BlockSpec((_BN, H), blk),
            pl.BlockSpec((_BN, 1), blk),
        ],
        out_shape=[
            jax.ShapeDtypeStruct((N, H), jnp.float32),
            jax.ShapeDtypeStruct((N, H), jnp.float32),
            jax.ShapeDtypeStruct((N, H), jnp.float32),
            jax.ShapeDtypeStruct((N, 1), jnp.float32),
        ],
    )(x, tf, s0, s1, c0, c1, dt, wf1x, wf2, bf2, wg2, bg2, wg1a, wg1b, wo, bo)


def kernel(x0, t, edge_index, W_e1, b_e1, W_e2, b_e2, theta, w,
           WF1, bF1, WF2, bF2, WG1, bG1, WG2, bG2, Wo, bo):
    src = edge_index[0].astype(jnp.int32)
    dst = edge_index[1].astype(jnp.int32)

    wg1a = WG1[:H]
    wg1b = WG1[H:2 * H]
    wg1w = WG1[2 * H:]
    wf1x = WF1[:H]
    wf1t = WF1[H:]

    be1 = b_e1.reshape(1, H)
    be2 = b_e2.reshape(1, H)
    bf1 = bF1.reshape(1, H)
    bf2 = bF2.reshape(1, H)
    bg1 = bG1.reshape(1, H)
    bg2 = bG2.reshape(1, H)
    bo2 = bo.reshape(1, 1)

    wq_b = _wq_call(w, wg1w[:, _SIG], bg1[:, _SIG])
    wq = jax.lax.bitcast_convert_type(
        wq_b.reshape(E, H // 2, 2), jnp.int32)
    cnt = _cnt_call(dst, jnp.ones((_BE, 16), jnp.float32),
                    jnp.zeros((_RPT, 16), jnp.float32))
    c0, c1 = cnt[0, :N], cnt[1, :N]

    x, xa, xb, tf, y0 = _k0_call(
        x0.reshape(N, 1), W_e1, be1, W_e2, be2, theta, wf1t, bf1,
        wg1a, wg1b, Wo, bo2)

    zeros_h = jnp.zeros((_RPT, H), jnp.float32)
    ys = [y0]
    for i in range(T - 1):
        dt = (t[i + 1] - t[i]).reshape(1, 1)
        s = _edge_call(src, dst, xa, xb, wq, zeros_h)
        x, xa, xb, y = _k1_call(x, tf, s[0, :N], s[1, :N], c0, c1, dt, wf1x,
                                WF2, bf2, WG2, bg2, wg1a, wg1b, Wo, bo2)
        ys.append(y)

    return jnp.concatenate(ys, axis=1).T


# speedup vs baseline: 2.0849x; 2.0849x over previous
import functools

import numpy as np
import jax
import jax.numpy as jnp
from jax import lax
from jax.experimental import pallas as pl
from jax.experimental.pallas import tpu as pltpu
from jax.experimental.pallas import tpu_sc as plsc

N = 10000
E = 320000
H = 128
T = 5

_NC = 2
_NS = 16
_NW = _NC * _NS
_EW = E // _NW
_BE = 40
_NBLK = _EW // _BE
_NP = 10240
_RPT = _NP // _NS
_LG = H // 16

_mesh = plsc.VectorSubcoreMesh(core_axis_name="c", subcore_axis_name="s")


def _edge_body(src_h, dst_h, xa_h, xb_h, wq_h, zeros_h, out_h,
               sidx0, sidx1, didx0, didx1, didx2, didx3,
               ra0, ra1, rb0, rb1, rw0, rw1,
               rt0, rt1, s_sh, gsem0, gsem1, ssem0, ssem1, isem0, isem1):
    c = lax.axis_index("c")
    s = lax.axis_index("s")
    wid = s * _NC + c
    base = wid * _EW

    sidx = (sidx0, sidx1)
    didx = (didx0, didx1, didx2, didx3)
    ra = (ra0, ra1)
    rb = (rb0, rb1)
    rw = (rw0, rw1)
    rt = (rt0, rt1)
    gsem = (gsem0, gsem1)
    ssem = (ssem0, ssem1)
    isem = (isem0, isem1)

    pltpu.sync_copy(zeros_h, s_sh.at[pl.ds(s * _RPT, _RPT)])
    plsc.subcore_barrier()

    def start_block(b, p, j):
        off = base + b * _BE
        pltpu.sync_copy(src_h.at[pl.ds(off, _BE)], sidx[p])
        pltpu.sync_copy(dst_h.at[pl.ds(off, _BE)], didx[j])
        pltpu.async_copy(xa_h.at[didx[j]], ra[p], gsem[p])
        pltpu.async_copy(xb_h.at[sidx[p]], rb[p], gsem[p])
        pltpu.async_copy(wq_h.at[pl.ds(off, _BE)], rw[p], gsem[p])

    def wait_scatter(p, j):
        pltpu.make_async_copy(rt[p], s_sh.at[didx[j]], ssem[p]).wait()

    def do_phase(b, p, j, first=False, last=False):
        pltpu.make_async_copy(xa_h.at[didx[j]], ra[p], gsem[p]).wait()
        pltpu.make_async_copy(xb_h.at[sidx[p]], rb[p], gsem[p]).wait()
        pltpu.make_async_copy(wq_h.at[pl.ds(0, _BE)], rw[p], gsem[p]).wait()
        if not first:
            wait_scatter(p, (j + 2) % 4)
        if not last:
            off = base + (b + 2) * _BE
            pltpu.async_copy(src_h.at[pl.ds(off, _BE)], sidx[p], isem[p])
            pltpu.async_copy(dst_h.at[pl.ds(off, _BE)], didx[(j + 2) % 4],
                             isem[p])

        cra, crb, crw, crt = ra[p], rb[p], rw[p], rt[p]

        def edge(e, cc):
            for g in range(_LG):
                sl = pl.ds(16 * g, 16)
                z = cra[e, sl] + crb[e, sl] + crw[e, sl]
                u = jnp.exp(z + z)
                crt[e, sl] = 1.0 - 2.0 / (u + 1.0)
            return cc

        lax.fori_loop(0, _BE, edge, 0)
        pltpu.async_copy(rt[p], s_sh.at[didx[j]], ssem[p], add=True)
        if not last:
            jj = (j + 2) % 4
            off = base + (b + 2) * _BE
            pltpu.make_async_copy(src_h.at[pl.ds(0, _BE)], sidx[p],
                                  isem[p]).wait()
            pltpu.make_async_copy(dst_h.at[pl.ds(0, _BE)], didx[jj],
                                  isem[p]).wait()
            pltpu.async_copy(xa_h.at[didx[jj]], ra[p], gsem[p])
            pltpu.async_copy(xb_h.at[sidx[p]], rb[p], gsem[p])
            pltpu.async_copy(wq_h.at[pl.ds(off, _BE)], rw[p], gsem[p])

    start_block(0, 0, 0)
    start_block(1, 1, 1)
    do_phase(0, 0, 0, first=True)
    do_phase(1, 1, 1, first=True)

    def quad(k, carry):
        b0 = 2 + 4 * k
        do_phase(b0, 0, 2)
        do_phase(b0 + 1, 1, 3)
        do_phase(b0 + 2, 0, 0)
        do_phase(b0 + 3, 1, 1)
        return carry

    lax.fori_loop(0, (_NBLK - 6) // 4, quad, 0)
    do_phase(_NBLK - 4, 0, 2)
    do_phase(_NBLK - 3, 1, 3)
    do_phase(_NBLK - 2, 0, 0, last=True)
    do_phase(_NBLK - 1, 1, 1, last=True)
    wait_scatter(0, 0)
    wait_scatter(1, 1)

    plsc.subcore_barrier()
    pltpu.sync_copy(s_sh.at[pl.ds(s * _RPT, _RPT)],
                    out_h.at[c, pl.ds(s * _RPT, _RPT)])


_edge_call = functools.partial(
    pl.kernel, _edge_body,
    out_type=jax.ShapeDtypeStruct((_NC, _NP, H), jnp.float32),
    mesh=_mesh,
    scratch_types=[
        pltpu.VMEM((_BE,), jnp.int32),
        pltpu.VMEM((_BE,), jnp.int32),
        pltpu.VMEM((_BE,), jnp.int32),
        pltpu.VMEM((_BE,), jnp.int32),
        pltpu.VMEM((_BE,), jnp.int32),
        pltpu.VMEM((_BE,), jnp.int32),
        pltpu.VMEM((_BE, H), jnp.float32),
        pltpu.VMEM((_BE, H), jnp.float32),
        pltpu.VMEM((_BE, H), jnp.float32),
        pltpu.VMEM((_BE, H), jnp.float32),
        pltpu.VMEM((_BE, H), jnp.float32),
        pltpu.VMEM((_BE, H), jnp.float32),
        pltpu.VMEM((_BE, H), jnp.float32),
        pltpu.VMEM((_BE, H), jnp.float32),
        pltpu.VMEM_SHARED((_NP, H), jnp.float32),
        pltpu.SemaphoreType.DMA,
        pltpu.SemaphoreType.DMA,
        pltpu.SemaphoreType.DMA,
        pltpu.SemaphoreType.DMA,
        pltpu.SemaphoreType.DMA,
        pltpu.SemaphoreType.DMA,
    ],
)()


_BEC = 80
_NBC = _EW // _BEC


def _cnt_body(dst_h, ones_h, zeros_h, out_h,
              didx0, didx1, didx2, didx3, ones_v, c_sh, ssem0, ssem1):
    c = lax.axis_index("c")
    s = lax.axis_index("s")
    wid = s * _NC + c
    base = wid * _EW

    didx = (didx0, didx1, didx2, didx3)
    ssem = (ssem0, ssem1)

    pltpu.sync_copy(zeros_h, c_sh.at[pl.ds(s * _RPT, _RPT)])
    pltpu.sync_copy(ones_h, ones_v)
    plsc.subcore_barrier()

    def phase(b, p, j, first=False, last=False):
        if not first:
            pltpu.make_async_copy(ones_v, c_sh.at[didx[(j + 2) % 4]],
                                  ssem[p]).wait()
        pltpu.async_copy(ones_v, c_sh.at[didx[j]], ssem[p], add=True)
        if not last:
            off = base + (b + 2) * _BEC
            pltpu.sync_copy(dst_h.at[pl.ds(off, _BEC)], didx[(j + 2) % 4])

    pltpu.sync_copy(dst_h.at[pl.ds(base, _BEC)], didx0)
    pltpu.sync_copy(dst_h.at[pl.ds(base + _BEC, _BEC)], didx1)
    phase(0, 0, 0, first=True)
    phase(1, 1, 1, first=True)

    def quad(k, carry):
        b0 = 2 + 4 * k
        phase(b0, 0, 2)
        phase(b0 + 1, 1, 3)
        phase(b0 + 2, 0, 0)
        phase(b0 + 3, 1, 1)
        return carry

    lax.fori_loop(0, (_NBC - 5) // 4, quad, 0)
    phase(_NBC - 3, 0, 2)
    phase(_NBC - 2, 1, 3, last=True)
    phase(_NBC - 1, 0, 0, last=True)
    pltpu.make_async_copy(ones_v, c_sh.at[didx0], ssem[0]).wait()
    pltpu.make_async_copy(ones_v, c_sh.at[didx1], ssem[1]).wait()

    plsc.subcore_barrier()
    pltpu.sync_copy(c_sh.at[pl.ds(s * _RPT, _RPT)],
                    out_h.at[c, pl.ds(s * _RPT, _RPT)])


_cnt_call = functools.partial(
    pl.kernel, _cnt_body,
    out_type=jax.ShapeDtypeStruct((_NC, _NP, 16), jnp.float32),
    mesh=_mesh,
    scratch_types=[
        pltpu.VMEM((_BEC,), jnp.int32),
        pltpu.VMEM((_BEC,), jnp.int32),
        pltpu.VMEM((_BEC,), jnp.int32),
        pltpu.VMEM((_BEC,), jnp.int32),
        pltpu.VMEM((_BEC, 16), jnp.float32),
        pltpu.VMEM_SHARED((_NP, 16), jnp.float32),
        pltpu.SemaphoreType.DMA,
        pltpu.SemaphoreType.DMA,
    ],
)()


_BEW = 4000
_BN = 2000


def _wq_body(w_ref, wg_ref, bg_ref, o_ref):
    o_ref[...] = (
        jnp.dot(w_ref[...], wg_ref[...], preferred_element_type=jnp.float32)
        + bg_ref[...]
    )


def _wq_call(w, wg1w, bg1):
    return pl.pallas_call(
        _wq_body,
        grid=(E // _BEW,),
        in_specs=[
            pl.BlockSpec((_BEW, H), lambda i: (i, 0)),
            pl.BlockSpec((H, H), lambda i: (0, 0)),
            pl.BlockSpec((1, H), lambda i: (0, 0)),
        ],
        out_specs=pl.BlockSpec((_BEW, H), lambda i: (i, 0)),
        out_shape=jax.ShapeDtypeStruct((E, H), jnp.float32),
    )(w, wg1w, bg1)


def _k0_body(x01_ref, we1_ref, be1_ref, we2_ref, be2_ref, th_ref, wf1t_ref,
             bf1_ref, wg1a_ref, wg1b_ref, wo_ref, bo_ref,
             x_ref, xa_ref, xb_ref, tf_ref, y0_ref):
    h1 = jnp.tanh(x01_ref[...] * we1_ref[...] + be1_ref[...])
    x = jnp.dot(h1, we2_ref[...], preferred_element_type=jnp.float32) + be2_ref[...]
    x_ref[...] = x
    xa_ref[...] = jnp.dot(x, wg1a_ref[...], preferred_element_type=jnp.float32)
    xb_ref[...] = jnp.dot(x, wg1b_ref[...], preferred_element_type=jnp.float32)
    tf_ref[...] = (
        jnp.dot(th_ref[...], wf1t_ref[...], preferred_element_type=jnp.float32)
        + bf1_ref[...]
    )
    y0_ref[...] = jnp.dot(x, wo_ref[...], preferred_element_type=jnp.float32) + bo_ref[...]


def _k0_call(x01, we1, be1, we2, be2, theta, wf1t, bf1, wg1a, wg1b, wo, bo):
    full = lambda i: (0, 0)
    blk = lambda i: (i, 0)
    return pl.pallas_call(
        _k0_body,
        grid=(N // _BN,),
        in_specs=[
            pl.BlockSpec((_BN, 1), blk),
            pl.BlockSpec((1, H), full),
            pl.BlockSpec((1, H), full),
            pl.BlockSpec((H, H), full),
            pl.BlockSpec((1, H), full),
            pl.BlockSpec((_BN, H), blk),
            pl.BlockSpec((H, H), full),
            pl.BlockSpec((1, H), full),
            pl.BlockSpec((H, H), full),
            pl.BlockSpec((H, H), full),
            pl.BlockSpec((H, 1), full),
            pl.BlockSpec((1, 1), full),
        ],
        out_specs=[
            pl.BlockSpec((_BN, H), blk),
            pl.BlockSpec((_BN, H), blk),
            pl.BlockSpec((_BN, H), blk),
            pl.BlockSpec((_BN, H), blk),
            pl.BlockSpec((_BN, 1), blk),
        ],
        out_shape=[
            jax.ShapeDtypeStruct((N, H), jnp.float32),
            jax.ShapeDtypeStruct((N, H), jnp.float32),
            jax.ShapeDtypeStruct((N, H), jnp.float32),
            jax.ShapeDtypeStruct((N, H), jnp.float32),
            jax.ShapeDtypeStruct((N, 1), jnp.float32),
        ],
    )(x01, we1, be1, we2, be2, theta, wf1t, bf1, wg1a, wg1b, wo, bo)


def _k1_body(x_ref, tf_ref, s0_ref, s1_ref, c0_ref, c1_ref, dt_ref,
             wf1x_ref, wf2_ref, bf2_ref, wg2_ref, bg2_ref, wg1a_ref,
             wg1b_ref, wo_ref, bo_ref,
             xn_ref, xa_ref, xb_ref, y_ref):
    x = x_ref[...]
    h = jnp.tanh(
        jnp.dot(x, wf1x_ref[...], preferred_element_type=jnp.float32) + tf_ref[...]
    )
    self_h = jnp.dot(h, wf2_ref[...], preferred_element_type=jnp.float32) + bf2_ref[...]
    s = s0_ref[...] + s1_ref[...]
    cnt = c0_ref[:, :1] + c1_ref[:, :1]
    agg = (
        jnp.dot(s, wg2_ref[...], preferred_element_type=jnp.float32)
        + cnt * bg2_ref[...]
    )
    xn = x + dt_ref[0, 0] * (self_h + agg)
    xn_ref[...] = xn
    xa_ref[...] = jnp.dot(xn, wg1a_ref[...], preferred_element_type=jnp.float32)
    xb_ref[...] = jnp.dot(xn, wg1b_ref[...], preferred_element_type=jnp.float32)
    y_ref[...] = jnp.dot(xn, wo_ref[...], preferred_element_type=jnp.float32) + bo_ref[...]


def _k1_call(x, tf, s0, s1, c0, c1, dt, wf1x, wf2, bf2, wg2, bg2, wg1a,
             wg1b, wo, bo):
    full = lambda i: (0, 0)
    blk = lambda i: (i, 0)
    return pl.pallas_call(
        _k1_body,
        grid=(N // _BN,),
        in_specs=[
            pl.BlockSpec((_BN, H), blk),
            pl.BlockSpec((_BN, H), blk),
            pl.BlockSpec((_BN, H), blk),
            pl.BlockSpec((_BN, H), blk),
            pl.BlockSpec((_BN, 16), blk),
            pl.BlockSpec((_BN, 16), blk),
            pl.BlockSpec((1, 1), full),
            pl.BlockSpec((H, H), full),
            pl.BlockSpec((H, H), full),
            pl.BlockSpec((1, H), full),
            pl.BlockSpec((H, H), full),
            pl.BlockSpec((1, H), full),
            pl.BlockSpec((H, H), full),
            pl.BlockSpec((H, H), full),
            pl.BlockSpec((H, 1), full),
            pl.BlockSpec((1, 1), full),
        ],
        out_specs=[
            pl.BlockSpec((_BN, H), blk),
            pl.BlockSpec((_BN, H), blk),
            pl.BlockSpec((_BN, H), blk),
            pl.BlockSpec((_BN, 1), blk),
        ],
        out_shape=[
            jax.ShapeDtypeStruct((N, H), jnp.float32),
            jax.ShapeDtypeStruct((N, H), jnp.float32),
            jax.ShapeDtypeStruct((N, H), jnp.float32),
            jax.ShapeDtypeStruct((N, 1), jnp.float32),
        ],
    )(x, tf, s0, s1, c0, c1, dt, wf1x, wf2, bf2, wg2, bg2, wg1a, wg1b, wo, bo)


def kernel(x0, t, edge_index, W_e1, b_e1, W_e2, b_e2, theta, w,
           WF1, bF1, WF2, bF2, WG1, bG1, WG2, bG2, Wo, bo):
    src = edge_index[0].astype(jnp.int32)
    dst = edge_index[1].astype(jnp.int32)

    wg1a = WG1[:H]
    wg1b = WG1[H:2 * H]
    wg1w = WG1[2 * H:]
    wf1x = WF1[:H]
    wf1t = WF1[H:]

    be1 = b_e1.reshape(1, H)
    be2 = b_e2.reshape(1, H)
    bf1 = bF1.reshape(1, H)
    bf2 = bF2.reshape(1, H)
    bg1 = bG1.reshape(1, H)
    bg2 = bG2.reshape(1, H)
    bo2 = bo.reshape(1, 1)

    wq = _wq_call(w, wg1w, bg1)
    cnt = _cnt_call(dst, jnp.ones((_BEC, 16), jnp.float32),
                    jnp.zeros((_RPT, 16), jnp.float32))
    c0, c1 = cnt[0, :N], cnt[1, :N]

    x, xa, xb, tf, y0 = _k0_call(
        x0.reshape(N, 1), W_e1, be1, W_e2, be2, theta, wf1t, bf1,
        wg1a, wg1b, Wo, bo2)

    zeros_h = jnp.zeros((_RPT, H), jnp.float32)
    ys = [y0]
    for i in range(T - 1):
        dt = (t[i + 1] - t[i]).reshape(1, 1)
        s = _edge_call(src, dst, xa, xb, wq, zeros_h)
        x, xa, xb, y = _k1_call(x, tf, s[0, :N], s[1, :N], c0, c1, dt, wf1x,
                                WF2, bf2, WG2, bg2, wg1a, wg1b, Wo, bo2)
        ys.append(y)

    return jnp.concatenate(ys, axis=1).T
